# Initial kernel scaffold; baseline (speedup 1.0000x reference)
import jax
import jax.numpy as jnp
from jax.experimental import pallas as pl


def kernel(x, edge_index, lin_W, lin_b, prelu_a, conv1_W, conv1_b, conv2_W, conv2_b):
    raise NotImplementedError("write your pallas kernel here")


# speedup vs baseline: 1.0607x; 1.0607x over previous
import functools

import jax
import jax.numpy as jnp
from jax.experimental import pallas as pl
from jax.experimental.pallas import tpu as pltpu

N = 10000
E = 320000


def _dense_body(x_ref, w_ref, b_ref, a_ref, o_ref):
    h = jnp.dot(x_ref[...], w_ref[...], preferred_element_type=jnp.float32)
    h = h + b_ref[...]
    a = a_ref[0]
    o_ref[...] = jnp.where(h >= 0, h, a * h)


def _dense_prelu_matmul(x, W, b, a):
    M, K = x.shape
    O = W.shape[0]
    BM = 1000
    grid = (M // BM,)
    return pl.pallas_call(
        _dense_body,
        grid=grid,
        in_specs=[
            pl.BlockSpec((BM, K), lambda i: (i, 0)),
            pl.BlockSpec((K, O), lambda i: (0, 0)),
            pl.BlockSpec((1, O), lambda i: (0, 0)),
            pl.BlockSpec(memory_space=pltpu.SMEM),
        ],
        out_specs=pl.BlockSpec((BM, O), lambda i: (i, 0)),
        out_shape=jax.ShapeDtypeStruct((M, O), jnp.float32),
    )(x, W.T, b[None, :], a.reshape(1))


def kernel(x, edge_index, lin_W, lin_b, prelu_a, conv1_W, conv1_b, conv2_W, conv2_b):
    src = edge_index[0]
    dst = edge_index[1]

    h = _dense_prelu_matmul(x, lin_W, lin_b, prelu_a)

    loop = jnp.arange(N, dtype=src.dtype)
    s = jnp.concatenate([src, loop])
    d = jnp.concatenate([dst, loop])
    deg = jnp.zeros((N,), dtype=x.dtype).at[d].add(1.0)
    dinv = jnp.where(deg > 0, 1.0 / jnp.sqrt(deg), 0.0)
    norm = dinv[s] * dinv[d]

    for W, b in ((conv1_W, conv1_b), (conv2_W, conv2_b)):
        g = h @ W.T
        msg = g[s] * norm[:, None]
        out = jnp.zeros((N, W.shape[0]), dtype=x.dtype).at[d].add(msg)
        pre = out + b
        h = jnp.where(pre >= 0, pre, prelu_a * pre)
    return h


# speedup vs baseline: 14.6342x; 13.7967x over previous
import jax
import jax.numpy as jnp
from jax import lax
from jax.experimental import pallas as pl
from jax.experimental.pallas import tpu as pltpu
from jax.experimental.pallas import tpu_sc as plsc

N = 10000
E = 320000
IN_DIM = 128
HID = 256
OUT = 256
HALF = 128

CHUNK = 125
NCHUNK = E // CHUNK
CPT = NCHUNK // 16
SB = CPT // 8

_mesh = plsc.VectorSubcoreMesh(core_axis_name="c", subcore_axis_name="s")


def _fill(ref, nrows, value):
    v = jnp.full((16,), value, dtype=jnp.float32)
    cols = ref.shape[1] // 16

    def body(j, _):
        r = j // cols
        c = j % cols
        ref[r, pl.ds(c * 16, 16)] = v
        return 0

    lax.fori_loop(0, nrows * cols, body, 0)


def _tile_rows(sid):
    return sid * 624


def _deg_body(edges_d, out_deg, edbuf, buf, dacc):
    cid = lax.axis_index("c")
    sid = lax.axis_index("s")
    base = _tile_rows(sid)

    _fill(buf, CHUNK, 0.0)
    for k in range(5):
        pltpu.sync_copy(buf.at[pl.ds(0, 120)], dacc.at[pl.ds(base + 120 * k, 120)])

    @pl.when(sid < 15)
    def _():
        pltpu.sync_copy(buf.at[pl.ds(0, 24)], dacc.at[pl.ds(base + 600, 24)])

    @pl.when(sid == 15)
    def _():
        pltpu.sync_copy(buf.at[pl.ds(0, 40)], dacc.at[pl.ds(base + 600, 40)])

    plsc.subcore_barrier()

    _fill(buf, CHUNK, 1.0)
    cbase = sid * CPT

    def body(b, _):
        pltpu.sync_copy(edges_d.at[pl.ds(cbase + b * 8, 8)], edbuf)
        for j in range(8):
            pltpu.sync_copy(buf, dacc.at[edbuf.at[j]], add=True)
        return 0

    lax.fori_loop(0, SB, body, 0)
    plsc.subcore_barrier()

    @pl.when(cid == 0)
    def _():
        for k in range(5):
            pltpu.sync_copy(dacc.at[pl.ds(base + 120 * k, 120)],
                            out_deg.at[pl.ds(base + 120 * k, 120)])

        @pl.when(sid < 15)
        def _():
            pltpu.sync_copy(dacc.at[pl.ds(base + 600, 24)],
                            out_deg.at[pl.ds(base + 600, 24)])

        @pl.when(sid == 15)
        def _():
            pltpu.sync_copy(dacc.at[pl.ds(base + 600, 40)],
                            out_deg.at[pl.ds(base + 600, 40)])


_deg_call = pl.kernel(
    _deg_body,
    out_type=jax.ShapeDtypeStruct((N, 16), jnp.float32),
    mesh=_mesh,
    scratch_types=[
        pltpu.VMEM((8, CHUNK), jnp.int32),
        pltpu.VMEM((CHUNK, 16), jnp.float32),
        pltpu.VMEM_SHARED((N, 16), jnp.float32),
    ],
)


def _agg_body(edges_s, edges_d, gs_lo, gs_hi, out_lo, out_hi,
              esbuf, edbuf, rows, acc):
    cid = lax.axis_index("c")
    sid = lax.axis_index("s")
    base = _tile_rows(sid)

    def run(gs_ref, out_ref):
        _fill(rows, CHUNK, 0.0)
        for k in range(5):
            pltpu.sync_copy(rows.at[pl.ds(0, 120)],
                            acc.at[pl.ds(base + 120 * k, 120)])

        @pl.when(sid < 15)
        def _():
            pltpu.sync_copy(rows.at[pl.ds(0, 24)], acc.at[pl.ds(base + 600, 24)])

        @pl.when(sid == 15)
        def _():
            pltpu.sync_copy(rows.at[pl.ds(0, 40)], acc.at[pl.ds(base + 600, 40)])

        plsc.subcore_barrier()

        cbase = sid * CPT

        def body(b, _):
            pltpu.sync_copy(edges_s.at[pl.ds(cbase + b * 8, 8)], esbuf)
            pltpu.sync_copy(edges_d.at[pl.ds(cbase + b * 8, 8)], edbuf)
            for j in range(8):
                pltpu.sync_copy(gs_ref.at[esbuf.at[j]], rows)
                pltpu.sync_copy(rows, acc.at[edbuf.at[j]], add=True)
            return 0

        lax.fori_loop(0, SB, body, 0)
        plsc.subcore_barrier()

        for k in range(5):
            pltpu.sync_copy(acc.at[pl.ds(base + 120 * k, 120)],
                            out_ref.at[pl.ds(base + 120 * k, 120)])

        @pl.when(sid < 15)
        def _():
            pltpu.sync_copy(acc.at[pl.ds(base + 600, 24)],
                            out_ref.at[pl.ds(base + 600, 24)])

        @pl.when(sid == 15)
        def _():
            pltpu.sync_copy(acc.at[pl.ds(base + 600, 40)],
                            out_ref.at[pl.ds(base + 600, 40)])

    @pl.when(cid == 0)
    def _():
        run(gs_lo, out_lo)

    @pl.when(cid == 1)
    def _():
        run(gs_hi, out_hi)


_agg_call = pl.kernel(
    _agg_body,
    out_type=[jax.ShapeDtypeStruct((N, HALF), jnp.float32),
              jax.ShapeDtypeStruct((N, HALF), jnp.float32)],
    mesh=_mesh,
    scratch_types=[
        pltpu.VMEM((8, CHUNK), jnp.int32),
        pltpu.VMEM((8, CHUNK), jnp.int32),
        pltpu.VMEM((CHUNK, HALF), jnp.float32),
        pltpu.VMEM_SHARED((N, HALF), jnp.float32),
    ],
)


BM = 2000


def _stage_b_body(x_ref, w1t, b1, a_ref, w2t, deg_ref, g1_ref, lo_ref, hi_ref):
    a = a_ref[0]
    h0 = jnp.dot(x_ref[...], w1t[...], preferred_element_type=jnp.float32)
    h0 = h0 + b1[...]
    h0 = jnp.where(h0 >= 0, h0, a * h0)
    g1 = jnp.dot(h0, w2t[...], preferred_element_type=jnp.float32)
    dinv = lax.rsqrt(deg_ref[:, 0:1] + 1.0)
    gs = g1 * dinv
    g1_ref[...] = g1
    lo_ref[...] = gs[:, :HALF]
    hi_ref[...] = gs[:, HALF:]


def _stage_d_body(lo_in, hi_in, g_in, b_ref, a_ref, wt, deg_ref,
                  g2_ref, lo_ref, hi_ref):
    a = a_ref[0]
    dinv = lax.rsqrt(deg_ref[:, 0:1] + 1.0)
    scat = jnp.concatenate([lo_in[...], hi_in[...]], axis=1)
    pre = scat * dinv + g_in[...] * (dinv * dinv) + b_ref[...]
    h = jnp.where(pre >= 0, pre, a * pre)
    g2 = jnp.dot(h, wt[...], preferred_element_type=jnp.float32)
    gs = g2 * dinv
    g2_ref[...] = g2
    lo_ref[...] = gs[:, :HALF]
    hi_ref[...] = gs[:, HALF:]


def _stage_f_body(lo_in, hi_in, g_in, b_ref, a_ref, deg_ref, out_ref):
    a = a_ref[0]
    dinv = lax.rsqrt(deg_ref[:, 0:1] + 1.0)
    scat = jnp.concatenate([lo_in[...], hi_in[...]], axis=1)
    pre = scat * dinv + g_in[...] * (dinv * dinv) + b_ref[...]
    out_ref[...] = jnp.where(pre >= 0, pre, a * pre)


def _row_spec(d):
    return pl.BlockSpec((BM, d), lambda i: (i, 0))


def _full_spec(r, c):
    return pl.BlockSpec((r, c), lambda i: (0, 0))


_SMEM = pl.BlockSpec(memory_space=pltpu.SMEM)

_stage_b = pl.pallas_call(
    _stage_b_body,
    grid=(N // BM,),
    in_specs=[_row_spec(IN_DIM), _full_spec(IN_DIM, HID), _full_spec(1, HID),
              _SMEM, _full_spec(HID, OUT), _row_spec(16)],
    out_specs=[_row_spec(OUT), _row_spec(HALF), _row_spec(HALF)],
    out_shape=[jax.ShapeDtypeStruct((N, OUT), jnp.float32),
               jax.ShapeDtypeStruct((N, HALF), jnp.float32),
               jax.ShapeDtypeStruct((N, HALF), jnp.float32)],
)

_stage_d = pl.pallas_call(
    _stage_d_body,
    grid=(N // BM,),
    in_specs=[_row_spec(HALF), _row_spec(HALF), _row_spec(OUT),
              _full_spec(1, OUT), _SMEM, _full_spec(OUT, OUT), _row_spec(16)],
    out_specs=[_row_spec(OUT), _row_spec(HALF), _row_spec(HALF)],
    out_shape=[jax.ShapeDtypeStruct((N, OUT), jnp.float32),
               jax.ShapeDtypeStruct((N, HALF), jnp.float32),
               jax.ShapeDtypeStruct((N, HALF), jnp.float32)],
)

_stage_f = pl.pallas_call(
    _stage_f_body,
    grid=(N // BM,),
    in_specs=[_row_spec(HALF), _row_spec(HALF), _row_spec(OUT),
              _full_spec(1, OUT), _SMEM, _row_spec(16)],
    out_specs=_row_spec(OUT),
    out_shape=jax.ShapeDtypeStruct((N, OUT), jnp.float32),
)


def kernel(x, edge_index, lin_W, lin_b, prelu_a, conv1_W, conv1_b,
           conv2_W, conv2_b):
    edges_s = edge_index[0].reshape(NCHUNK, CHUNK)
    edges_d = edge_index[1].reshape(NCHUNK, CHUNK)
    a = prelu_a.reshape(1)

    deg = _deg_call(edges_d)
    g1, gs1lo, gs1hi = _stage_b(x, lin_W.T, lin_b[None], a, conv1_W.T, deg)
    agg1lo, agg1hi = _agg_call(edges_s, edges_d, gs1lo, gs1hi)
    g2, gs2lo, gs2hi = _stage_d(agg1lo, agg1hi, g1, conv1_b[None], a,
                                conv2_W.T, deg)
    agg2lo, agg2hi = _agg_call(edges_s, edges_d, gs2lo, gs2hi)
    return _stage_f(agg2lo, agg2hi, g2, conv2_b[None], a, deg)


# speedup vs baseline: 23.6978x; 1.6193x over previous
import jax
import jax.numpy as jnp
from jax import lax
from jax.experimental import pallas as pl
from jax.experimental.pallas import tpu as pltpu
from jax.experimental.pallas import tpu_sc as plsc

N = 10000
E = 320000
IN_DIM = 128
HID = 256
OUT = 256
HALF = 128

CHUNK = 125
NCHUNK = E // CHUNK
CPT = NCHUNK // 16
SB = CPT // 8
IBLK = 32

_mesh = plsc.VectorSubcoreMesh(core_axis_name="c", subcore_axis_name="s")


def _fill(ref, nrows, value):
    v = jnp.full((16,), value, dtype=jnp.float32)
    cols = ref.shape[1] // 16

    def body(j, _):
        r = j // cols
        c = j % cols
        ref[r, pl.ds(c * 16, 16)] = v
        return 0

    lax.fori_loop(0, nrows * cols, body, 0)


def _tile_rows(sid):
    return sid * 624


def _deg_body(edges_d, out_deg, edbuf, buf, dacc, isem, ssem):
    cid = lax.axis_index("c")
    sid = lax.axis_index("s")
    base = _tile_rows(sid)
    cbase = sid * CPT

    pltpu.async_copy(edges_d.at[pl.ds(cbase, CPT)], edbuf, isem)
    _fill(buf, CHUNK, 0.0)
    for k in range(5):
        pltpu.sync_copy(buf.at[pl.ds(0, 120)], dacc.at[pl.ds(base + 120 * k, 120)])

    @pl.when(sid < 15)
    def _():
        pltpu.sync_copy(buf.at[pl.ds(0, 24)], dacc.at[pl.ds(base + 600, 24)])

    @pl.when(sid == 15)
    def _():
        pltpu.sync_copy(buf.at[pl.ds(0, 40)], dacc.at[pl.ds(base + 600, 40)])

    _fill(buf, CHUNK, 1.0)
    pltpu.make_async_copy(edges_d.at[pl.ds(cbase, CPT)], edbuf, isem).wait()
    plsc.subcore_barrier()

    def body(b, _):
        for j in range(8):
            pltpu.async_copy(buf, dacc.at[edbuf.at[b * 8 + j]], ssem, add=True)
        for j in range(8):
            pltpu.make_async_copy(buf, dacc.at[edbuf.at[0]], ssem).wait()
        return 0

    lax.fori_loop(0, SB, body, 0)
    plsc.subcore_barrier()

    @pl.when(cid == 0)
    def _():
        for k in range(5):
            pltpu.sync_copy(dacc.at[pl.ds(base + 120 * k, 120)],
                            out_deg.at[pl.ds(base + 120 * k, 120)])

        @pl.when(sid < 15)
        def _():
            pltpu.sync_copy(dacc.at[pl.ds(base + 600, 24)],
                            out_deg.at[pl.ds(base + 600, 24)])

        @pl.when(sid == 15)
        def _():
            pltpu.sync_copy(dacc.at[pl.ds(base + 600, 40)],
                            out_deg.at[pl.ds(base + 600, 40)])


_deg_call = pl.kernel(
    _deg_body,
    out_type=jax.ShapeDtypeStruct((N, 16), jnp.float32),
    mesh=_mesh,
    scratch_types=[
        pltpu.VMEM((CPT, CHUNK), jnp.int32),
        pltpu.VMEM((CHUNK, 16), jnp.float32),
        pltpu.VMEM_SHARED((N, 16), jnp.float32),
        pltpu.SemaphoreType.DMA,
        pltpu.SemaphoreType.DMA,
    ],
)


def _agg_body(edges_s, edges_d, gs_lo, gs_hi, out_lo, out_hi,
              esbuf, edbuf, rows0, rows1, acc,
              gsem0, gsem1, ssem0, ssem1):
    cid = lax.axis_index("c")
    sid = lax.axis_index("s")
    base = _tile_rows(sid)
    cbase = sid * CPT

    def run(gs_ref, out_ref):
        _fill(rows0, CHUNK, 0.0)
        for k in range(5):
            pltpu.sync_copy(rows0.at[pl.ds(0, 120)],
                            acc.at[pl.ds(base + 120 * k, 120)])

        @pl.when(sid < 15)
        def _():
            pltpu.sync_copy(rows0.at[pl.ds(0, 24)], acc.at[pl.ds(base + 600, 24)])

        @pl.when(sid == 15)
        def _():
            pltpu.sync_copy(rows0.at[pl.ds(0, 40)], acc.at[pl.ds(base + 600, 40)])

        plsc.subcore_barrier()

        rows = (rows0, rows1)
        gsem = (gsem0, gsem1)
        ssem = (ssem0, ssem1)

        def g_start(j, b):
            pltpu.async_copy(gs_ref.at[esbuf.at[j]], rows[b], gsem[b])

        def g_wait(b):
            pltpu.make_async_copy(gs_ref.at[esbuf.at[0]], rows[b], gsem[b]).wait()

        def s_start(j, b):
            pltpu.async_copy(rows[b], acc.at[edbuf.at[j]], ssem[b], add=True)

        def s_wait(b):
            pltpu.make_async_copy(rows[b], acc.at[edbuf.at[0]], ssem[b]).wait()

        def blk(bi, _):
            pltpu.sync_copy(edges_s.at[pl.ds(cbase + IBLK * bi, IBLK)], esbuf)
            pltpu.sync_copy(edges_d.at[pl.ds(cbase + IBLK * bi, IBLK)], edbuf)
            g_start(0, 0)
            g_start(1, 1)
            g_wait(0)
            s_start(0, 0)

            def body(i, _):
                j = 2 * i
                s_wait(0)
                g_start(j, 0)
                g_wait(1)
                s_start(j - 1, 1)
                s_wait(1)
                g_start(j + 1, 1)
                g_wait(0)
                s_start(j, 0)
                return 0

            lax.fori_loop(1, IBLK // 2, body, 0)
            g_wait(1)
            s_start(IBLK - 1, 1)
            s_wait(0)
            s_wait(1)
            return 0

        lax.fori_loop(0, CPT // IBLK, blk, 0)
        plsc.subcore_barrier()

        for k in range(5):
            pltpu.sync_copy(acc.at[pl.ds(base + 120 * k, 120)],
                            out_ref.at[pl.ds(base + 120 * k, 120)])

        @pl.when(sid < 15)
        def _():
            pltpu.sync_copy(acc.at[pl.ds(base + 600, 24)],
                            out_ref.at[pl.ds(base + 600, 24)])

        @pl.when(sid == 15)
        def _():
            pltpu.sync_copy(acc.at[pl.ds(base + 600, 40)],
                            out_ref.at[pl.ds(base + 600, 40)])

    @pl.when(cid == 0)
    def _():
        run(gs_lo, out_lo)

    @pl.when(cid == 1)
    def _():
        run(gs_hi, out_hi)


_agg_call = pl.kernel(
    _agg_body,
    out_type=[jax.ShapeDtypeStruct((N, HALF), jnp.float32),
              jax.ShapeDtypeStruct((N, HALF), jnp.float32)],
    mesh=_mesh,
    scratch_types=[
        pltpu.VMEM((IBLK, CHUNK), jnp.int32),
        pltpu.VMEM((IBLK, CHUNK), jnp.int32),
        pltpu.VMEM((CHUNK, HALF), jnp.float32),
        pltpu.VMEM((CHUNK, HALF), jnp.float32),
        pltpu.VMEM_SHARED((N, HALF), jnp.float32),
        pltpu.SemaphoreType.DMA,
        pltpu.SemaphoreType.DMA,
        pltpu.SemaphoreType.DMA,
        pltpu.SemaphoreType.DMA,
    ],
)


BM = 2000


def _stage_b_body(x_ref, w1t, b1, a_ref, w2t, deg_ref, g1_ref, lo_ref, hi_ref):
    a = a_ref[0]
    h0 = jnp.dot(x_ref[...], w1t[...], preferred_element_type=jnp.float32)
    h0 = h0 + b1[...]
    h0 = jnp.where(h0 >= 0, h0, a * h0)
    g1 = jnp.dot(h0, w2t[...], preferred_element_type=jnp.float32)
    dinv = lax.rsqrt(deg_ref[:, 0:1] + 1.0)
    gs = g1 * dinv
    g1_ref[...] = g1
    lo_ref[...] = gs[:, :HALF]
    hi_ref[...] = gs[:, HALF:]


def _stage_d_body(lo_in, hi_in, g_in, b_ref, a_ref, wt, deg_ref,
                  g2_ref, lo_ref, hi_ref):
    a = a_ref[0]
    dinv = lax.rsqrt(deg_ref[:, 0:1] + 1.0)
    scat = jnp.concatenate([lo_in[...], hi_in[...]], axis=1)
    pre = scat * dinv + g_in[...] * (dinv * dinv) + b_ref[...]
    h = jnp.where(pre >= 0, pre, a * pre)
    g2 = jnp.dot(h, wt[...], preferred_element_type=jnp.float32)
    gs = g2 * dinv
    g2_ref[...] = g2
    lo_ref[...] = gs[:, :HALF]
    hi_ref[...] = gs[:, HALF:]


def _stage_f_body(lo_in, hi_in, g_in, b_ref, a_ref, deg_ref, out_ref):
    a = a_ref[0]
    dinv = lax.rsqrt(deg_ref[:, 0:1] + 1.0)
    scat = jnp.concatenate([lo_in[...], hi_in[...]], axis=1)
    pre = scat * dinv + g_in[...] * (dinv * dinv) + b_ref[...]
    out_ref[...] = jnp.where(pre >= 0, pre, a * pre)


def _row_spec(d):
    return pl.BlockSpec((BM, d), lambda i: (i, 0))


def _full_spec(r, c):
    return pl.BlockSpec((r, c), lambda i: (0, 0))


_SMEM = pl.BlockSpec(memory_space=pltpu.SMEM)

_stage_b = pl.pallas_call(
    _stage_b_body,
    grid=(N // BM,),
    in_specs=[_row_spec(IN_DIM), _full_spec(IN_DIM, HID), _full_spec(1, HID),
              _SMEM, _full_spec(HID, OUT), _row_spec(16)],
    out_specs=[_row_spec(OUT), _row_spec(HALF), _row_spec(HALF)],
    out_shape=[jax.ShapeDtypeStruct((N, OUT), jnp.float32),
               jax.ShapeDtypeStruct((N, HALF), jnp.float32),
               jax.ShapeDtypeStruct((N, HALF), jnp.float32)],
)

_stage_d = pl.pallas_call(
    _stage_d_body,
    grid=(N // BM,),
    in_specs=[_row_spec(HALF), _row_spec(HALF), _row_spec(OUT),
              _full_spec(1, OUT), _SMEM, _full_spec(OUT, OUT), _row_spec(16)],
    out_specs=[_row_spec(OUT), _row_spec(HALF), _row_spec(HALF)],
    out_shape=[jax.ShapeDtypeStruct((N, OUT), jnp.float32),
               jax.ShapeDtypeStruct((N, HALF), jnp.float32),
               jax.ShapeDtypeStruct((N, HALF), jnp.float32)],
)

_stage_f = pl.pallas_call(
    _stage_f_body,
    grid=(N // BM,),
    in_specs=[_row_spec(HALF), _row_spec(HALF), _row_spec(OUT),
              _full_spec(1, OUT), _SMEM, _row_spec(16)],
    out_specs=_row_spec(OUT),
    out_shape=jax.ShapeDtypeStruct((N, OUT), jnp.float32),
)


def kernel(x, edge_index, lin_W, lin_b, prelu_a, conv1_W, conv1_b,
           conv2_W, conv2_b):
    edges_s = edge_index[0].reshape(NCHUNK, CHUNK)
    edges_d = edge_index[1].reshape(NCHUNK, CHUNK)
    a = prelu_a.reshape(1)

    deg = _deg_call(edges_d)
    g1, gs1lo, gs1hi = _stage_b(x, lin_W.T, lin_b[None], a, conv1_W.T, deg)
    agg1lo, agg1hi = _agg_call(edges_s, edges_d, gs1lo, gs1hi)
    g2, gs2lo, gs2hi = _stage_d(agg1lo, agg1hi, g1, conv1_b[None], a,
                                conv2_W.T, deg)
    agg2lo, agg2hi = _agg_call(edges_s, edges_d, gs2lo, gs2hi)
    return _stage_f(agg2lo, agg2hi, g2, conv2_b[None], a, deg)


# speedup vs baseline: 23.7428x; 1.0019x over previous
import jax
import jax.numpy as jnp
from jax import lax
from jax.experimental import pallas as pl
from jax.experimental.pallas import tpu as pltpu
from jax.experimental.pallas import tpu_sc as plsc

N = 10000
E = 320000
IN_DIM = 128
HID = 256
OUT = 256
HALF = 128

CHUNK = 125
NCHUNK = E // CHUNK
CPT = NCHUNK // 16
SB = CPT // 8
IBLK = 32

_mesh = plsc.VectorSubcoreMesh(core_axis_name="c", subcore_axis_name="s")


def _fill(ref, nrows, value):
    v = jnp.full((16,), value, dtype=jnp.float32)
    cols = ref.shape[1] // 16

    def body(j, _):
        r = j // cols
        c = j % cols
        ref[r, pl.ds(c * 16, 16)] = v
        return 0

    lax.fori_loop(0, nrows * cols, body, 0)


def _tile_rows(sid):
    return sid * 624


def _deg_body(edges_d, out_deg, edbuf, buf, dacc, isem, ssem):
    cid = lax.axis_index("c")
    sid = lax.axis_index("s")
    base = _tile_rows(sid)
    cbase = sid * CPT

    pltpu.async_copy(edges_d.at[pl.ds(cbase, CPT)], edbuf, isem)
    _fill(buf, CHUNK, 0.0)
    for k in range(5):
        pltpu.sync_copy(buf.at[pl.ds(0, 120)], dacc.at[pl.ds(base + 120 * k, 120)])

    @pl.when(sid < 15)
    def _():
        pltpu.sync_copy(buf.at[pl.ds(0, 24)], dacc.at[pl.ds(base + 600, 24)])

    @pl.when(sid == 15)
    def _():
        pltpu.sync_copy(buf.at[pl.ds(0, 40)], dacc.at[pl.ds(base + 600, 40)])

    _fill(buf, CHUNK, 1.0)
    pltpu.make_async_copy(edges_d.at[pl.ds(cbase, CPT)], edbuf, isem).wait()
    plsc.subcore_barrier()

    def body(b, _):
        for j in range(8):
            pltpu.async_copy(buf, dacc.at[edbuf.at[b * 8 + j]], ssem, add=True)
        for j in range(8):
            pltpu.make_async_copy(buf, dacc.at[edbuf.at[0]], ssem).wait()
        return 0

    lax.fori_loop(0, SB, body, 0)
    plsc.subcore_barrier()

    @pl.when(cid == 0)
    def _():
        for k in range(5):
            pltpu.sync_copy(dacc.at[pl.ds(base + 120 * k, 120)],
                            out_deg.at[pl.ds(base + 120 * k, 120)])

        @pl.when(sid < 15)
        def _():
            pltpu.sync_copy(dacc.at[pl.ds(base + 600, 24)],
                            out_deg.at[pl.ds(base + 600, 24)])

        @pl.when(sid == 15)
        def _():
            pltpu.sync_copy(dacc.at[pl.ds(base + 600, 40)],
                            out_deg.at[pl.ds(base + 600, 40)])


_deg_call = pl.kernel(
    _deg_body,
    out_type=jax.ShapeDtypeStruct((N, 16), jnp.float32),
    mesh=_mesh,
    scratch_types=[
        pltpu.VMEM((CPT, CHUNK), jnp.int32),
        pltpu.VMEM((CHUNK, 16), jnp.float32),
        pltpu.VMEM_SHARED((N, 16), jnp.float32),
        pltpu.SemaphoreType.DMA,
        pltpu.SemaphoreType.DMA,
    ],
)


def _agg_body(edges_s, edges_d, gs_lo, gs_hi, out_lo, out_hi,
              esbuf, edbuf, rows0, rows1, acc,
              gsem0, gsem1, ssem0, ssem1):
    cid = lax.axis_index("c")
    sid = lax.axis_index("s")
    base = _tile_rows(sid)
    cbase = sid * CPT

    def run(gs_ref, out_ref):
        _fill(rows0, CHUNK, 0.0)
        for k in range(5):
            pltpu.sync_copy(rows0.at[pl.ds(0, 120)],
                            acc.at[pl.ds(base + 120 * k, 120)])

        @pl.when(sid < 15)
        def _():
            pltpu.sync_copy(rows0.at[pl.ds(0, 24)], acc.at[pl.ds(base + 600, 24)])

        @pl.when(sid == 15)
        def _():
            pltpu.sync_copy(rows0.at[pl.ds(0, 40)], acc.at[pl.ds(base + 600, 40)])

        plsc.subcore_barrier()

        rows = (rows0, rows1)
        gsem = (gsem0, gsem1)
        ssem = (ssem0, ssem1)

        def g_start(j, b):
            pltpu.async_copy(gs_ref.at[esbuf.at[j]], rows[b], gsem[b])

        def g_wait(b):
            pltpu.make_async_copy(gs_ref.at[esbuf.at[0]], rows[b], gsem[b]).wait()

        def s_start(j, b):
            pltpu.async_copy(rows[b], acc.at[edbuf.at[j]], ssem[b], add=True)

        def s_wait(b):
            pltpu.make_async_copy(rows[b], acc.at[edbuf.at[0]], ssem[b]).wait()

        def blk(bi, _):
            pltpu.sync_copy(edges_s.at[pl.ds(cbase + IBLK * bi, IBLK)], esbuf)
            pltpu.sync_copy(edges_d.at[pl.ds(cbase + IBLK * bi, IBLK)], edbuf)
            g_start(0, 0)
            g_start(1, 1)
            g_wait(0)
            s_start(0, 0)

            def body(i, _):
                j = 2 * i
                s_wait(0)
                g_start(j, 0)
                g_wait(1)
                s_start(j - 1, 1)
                s_wait(1)
                g_start(j + 1, 1)
                g_wait(0)
                s_start(j, 0)
                return 0

            lax.fori_loop(1, IBLK // 2, body, 0)
            g_wait(1)
            s_start(IBLK - 1, 1)
            s_wait(0)
            s_wait(1)
            return 0

        lax.fori_loop(0, CPT // IBLK, blk, 0)
        plsc.subcore_barrier()

        for k in range(5):
            pltpu.sync_copy(acc.at[pl.ds(base + 120 * k, 120)],
                            out_ref.at[pl.ds(base + 120 * k, 120)])

        @pl.when(sid < 15)
        def _():
            pltpu.sync_copy(acc.at[pl.ds(base + 600, 24)],
                            out_ref.at[pl.ds(base + 600, 24)])

        @pl.when(sid == 15)
        def _():
            pltpu.sync_copy(acc.at[pl.ds(base + 600, 40)],
                            out_ref.at[pl.ds(base + 600, 40)])

    @pl.when(cid == 0)
    def _():
        run(gs_lo, out_lo)

    @pl.when(cid == 1)
    def _():
        run(gs_hi, out_hi)


_agg_call = pl.kernel(
    _agg_body,
    out_type=[jax.ShapeDtypeStruct((N, HALF), jnp.float32),
              jax.ShapeDtypeStruct((N, HALF), jnp.float32)],
    mesh=_mesh,
    scratch_types=[
        pltpu.VMEM((IBLK, CHUNK), jnp.int32),
        pltpu.VMEM((IBLK, CHUNK), jnp.int32),
        pltpu.VMEM((CHUNK, HALF), jnp.float32),
        pltpu.VMEM((CHUNK, HALF), jnp.float32),
        pltpu.VMEM_SHARED((N, HALF), jnp.float32),
        pltpu.SemaphoreType.DMA,
        pltpu.SemaphoreType.DMA,
        pltpu.SemaphoreType.DMA,
        pltpu.SemaphoreType.DMA,
    ],
)


BM = 2000


def _stage_b1_body(x_ref, w1t, b1, a_ref, w2t, g1_ref):
    a = a_ref[0]
    h0 = jnp.dot(x_ref[...], w1t[...], preferred_element_type=jnp.float32)
    h0 = h0 + b1[...]
    h0 = jnp.where(h0 >= 0, h0, a * h0)
    g1_ref[...] = jnp.dot(h0, w2t[...], preferred_element_type=jnp.float32)


def _stage_b2_body(g1_ref, deg_ref, lo_ref, hi_ref):
    dinv = lax.rsqrt(deg_ref[:, 0:1] + 1.0)
    gs = g1_ref[...] * dinv
    lo_ref[...] = gs[:, :HALF]
    hi_ref[...] = gs[:, HALF:]


def _stage_d_body(lo_in, hi_in, g_in, b_ref, a_ref, wt, deg_ref,
                  g2_ref, lo_ref, hi_ref):
    a = a_ref[0]
    dinv = lax.rsqrt(deg_ref[:, 0:1] + 1.0)
    scat = jnp.concatenate([lo_in[...], hi_in[...]], axis=1)
    pre = scat * dinv + g_in[...] * (dinv * dinv) + b_ref[...]
    h = jnp.where(pre >= 0, pre, a * pre)
    g2 = jnp.dot(h, wt[...], preferred_element_type=jnp.float32)
    gs = g2 * dinv
    g2_ref[...] = g2
    lo_ref[...] = gs[:, :HALF]
    hi_ref[...] = gs[:, HALF:]


def _stage_f_body(lo_in, hi_in, g_in, b_ref, a_ref, deg_ref, out_ref):
    a = a_ref[0]
    dinv = lax.rsqrt(deg_ref[:, 0:1] + 1.0)
    scat = jnp.concatenate([lo_in[...], hi_in[...]], axis=1)
    pre = scat * dinv + g_in[...] * (dinv * dinv) + b_ref[...]
    out_ref[...] = jnp.where(pre >= 0, pre, a * pre)


def _row_spec(d):
    return pl.BlockSpec((BM, d), lambda i: (i, 0))


def _full_spec(r, c):
    return pl.BlockSpec((r, c), lambda i: (0, 0))


_SMEM = pl.BlockSpec(memory_space=pltpu.SMEM)

_stage_b1 = pl.pallas_call(
    _stage_b1_body,
    grid=(N // BM,),
    in_specs=[_row_spec(IN_DIM), _full_spec(IN_DIM, HID), _full_spec(1, HID),
              _SMEM, _full_spec(HID, OUT)],
    out_specs=_row_spec(OUT),
    out_shape=jax.ShapeDtypeStruct((N, OUT), jnp.float32),
)

_stage_b2 = pl.pallas_call(
    _stage_b2_body,
    grid=(N // BM,),
    in_specs=[_row_spec(OUT), _row_spec(16)],
    out_specs=[_row_spec(HALF), _row_spec(HALF)],
    out_shape=[jax.ShapeDtypeStruct((N, HALF), jnp.float32),
               jax.ShapeDtypeStruct((N, HALF), jnp.float32)],
)

_stage_d = pl.pallas_call(
    _stage_d_body,
    grid=(N // BM,),
    in_specs=[_row_spec(HALF), _row_spec(HALF), _row_spec(OUT),
              _full_spec(1, OUT), _SMEM, _full_spec(OUT, OUT), _row_spec(16)],
    out_specs=[_row_spec(OUT), _row_spec(HALF), _row_spec(HALF)],
    out_shape=[jax.ShapeDtypeStruct((N, OUT), jnp.float32),
               jax.ShapeDtypeStruct((N, HALF), jnp.float32),
               jax.ShapeDtypeStruct((N, HALF), jnp.float32)],
)

_stage_f = pl.pallas_call(
    _stage_f_body,
    grid=(N // BM,),
    in_specs=[_row_spec(HALF), _row_spec(HALF), _row_spec(OUT),
              _full_spec(1, OUT), _SMEM, _row_spec(16)],
    out_specs=_row_spec(OUT),
    out_shape=jax.ShapeDtypeStruct((N, OUT), jnp.float32),
)


def kernel(x, edge_index, lin_W, lin_b, prelu_a, conv1_W, conv1_b,
           conv2_W, conv2_b):
    edges_s = edge_index[0].reshape(NCHUNK, CHUNK)
    edges_d = edge_index[1].reshape(NCHUNK, CHUNK)
    a = prelu_a.reshape(1)

    deg = _deg_call(edges_d)
    g1 = _stage_b1(x, lin_W.T, lin_b[None], a, conv1_W.T)
    gs1lo, gs1hi = _stage_b2(g1, deg)
    agg1lo, agg1hi = _agg_call(edges_s, edges_d, gs1lo, gs1hi)
    g2, gs2lo, gs2hi = _stage_d(agg1lo, agg1hi, g1, conv1_b[None], a,
                                conv2_W.T, deg)
    agg2lo, agg2hi = _agg_call(edges_s, edges_d, gs2lo, gs2hi)
    return _stage_f(agg2lo, agg2hi, g2, conv2_b[None], a, deg)
